# Initial kernel scaffold; baseline (speedup 1.0000x reference)
import jax
import jax.numpy as jnp
from jax import lax
from jax.experimental import pallas as pl
from jax.experimental.pallas import tpu as pltpu, tpu_sc as plsc

N = 10000
NPAD = 10240
E = 160000
EPAD = 163840
D = 256
H = 128
TRASH = 10016
CK = 128
EPT = EPAD // 16
NCH = EPT // CK
RPT = NPAD // 16
XPT = NPAD // 32
XCK = 80

_mesh = plsc.VectorSubcoreMesh(core_axis_name="c", subcore_axis_name="s")
f32 = jnp.float32


def _prep_body(xp, emb, dstp, zeros1, h0, deg, deg_sh, rows_v, xi, di, ones_v,
               sem):
    c = lax.axis_index("c")
    s = lax.axis_index("s")
    wid = s * 2 + c
    for i in range(8):
        ones_v[pl.ds(i * 16, 16)] = jnp.ones((16,), f32)
    for j in range(XPT // XCK):
        off = wid * XPT + j * XCK
        pltpu.sync_copy(xp.at[pl.ds(off, XCK)], xi)
        pltpu.async_copy(emb.at[xi], rows_v, sem).wait()
        pltpu.sync_copy(rows_v, h0.at[pl.ds(off, XCK)])

    @pl.when(c == 0)
    def _():
        pltpu.sync_copy(zeros1.at[pl.ds(0, RPT)], deg_sh.at[pl.ds(s * RPT, RPT)])
    plsc.subcore_barrier()

    @pl.when(c == 0)
    def _():
        def body(j, carry):
            off = s * EPT + j * CK
            pltpu.sync_copy(dstp.at[pl.ds(off, CK)], di)
            pltpu.sync_copy(ones_v, deg_sh.at[di], add=True)
            return carry
        lax.fori_loop(0, NCH, body, 0)
    plsc.subcore_barrier()

    @pl.when(c == 0)
    def _():
        pltpu.sync_copy(deg_sh.at[pl.ds(s * RPT, RPT)], deg.at[pl.ds(s * RPT, RPT)])


_prep = pl.kernel(
    _prep_body,
    out_type=(jax.ShapeDtypeStruct((NPAD, D), f32),
              jax.ShapeDtypeStruct((NPAD,), f32)),
    mesh=_mesh,
    scratch_types=[pltpu.VMEM_SHARED((NPAD,), f32),
                   pltpu.VMEM((XCK, D), f32),
                   pltpu.VMEM((XCK,), jnp.int32),
                   pltpu.VMEM((CK,), jnp.int32),
                   pltpu.VMEM((CK,), f32),
                   pltpu.SemaphoreType.DMA],
)


def _msg_body(hwL, hwR, srcp, dstp, zeros_h, accL, accR, acc_sh, rows_v, si,
              di, sem):
    c = lax.axis_index("c")
    s = lax.axis_index("s")
    pltpu.sync_copy(zeros_h, acc_sh.at[pl.ds(s * RPT, RPT)])
    plsc.subcore_barrier()

    def body(j, carry):
        off = s * EPT + j * CK
        pltpu.sync_copy(srcp.at[pl.ds(off, CK)], si)
        pltpu.sync_copy(dstp.at[pl.ds(off, CK)], di)

        @pl.when(c == 0)
        def _():
            pltpu.async_copy(hwL.at[si], rows_v, sem).wait()

        @pl.when(c == 1)
        def _():
            pltpu.async_copy(hwR.at[si], rows_v, sem).wait()
        pltpu.sync_copy(rows_v, acc_sh.at[di], add=True)
        return carry
    lax.fori_loop(0, NCH, body, 0)
    plsc.subcore_barrier()

    @pl.when(c == 0)
    def _():
        pltpu.sync_copy(acc_sh.at[pl.ds(s * RPT, RPT)], accL.at[pl.ds(s * RPT, RPT)])

    @pl.when(c == 1)
    def _():
        pltpu.sync_copy(acc_sh.at[pl.ds(s * RPT, RPT)], accR.at[pl.ds(s * RPT, RPT)])


_msg = pl.kernel(
    _msg_body,
    out_type=(jax.ShapeDtypeStruct((NPAD, H), f32),
              jax.ShapeDtypeStruct((NPAD, H), f32)),
    mesh=_mesh,
    scratch_types=[pltpu.VMEM_SHARED((NPAD, H), f32),
                   pltpu.VMEM((CK, H), f32),
                   pltpu.VMEM((CK,), jnp.int32),
                   pltpu.VMEM((CK,), jnp.int32),
                   pltpu.SemaphoreType.DMA],
)


_GRID = NPAD // 256


def _k1_first_body(h_ref, w_ref, deg_ref, hl_ref, hr_ref, dv_ref):
    dinv = lax.rsqrt(deg_ref[...] + 1.0)
    hw = jnp.dot(h_ref[...], w_ref[...], preferred_element_type=f32)
    hl_ref[...] = hw[:, :H] * dinv
    hr_ref[...] = hw[:, H:] * dinv
    dv_ref[...] = dinv


def _k1_first(h0, W0, degb):
    return pl.pallas_call(
        _k1_first_body,
        grid=(_GRID,),
        in_specs=[pl.BlockSpec((256, D), lambda i: (i, 0)),
                  pl.BlockSpec((D, D), lambda i: (0, 0)),
                  pl.BlockSpec((256, H), lambda i: (i, 0))],
        out_specs=[pl.BlockSpec((256, H), lambda i: (i, 0)),
                   pl.BlockSpec((256, H), lambda i: (i, 0)),
                   pl.BlockSpec((256, H), lambda i: (i, 0))],
        out_shape=[jax.ShapeDtypeStruct((NPAD, H), f32),
                   jax.ShapeDtypeStruct((NPAD, H), f32),
                   jax.ShapeDtypeStruct((NPAD, H), f32)],
    )(h0, W0, degb)


def _k2_body(aL, aR, hL, hR, dv, b, z_ref, st_ref):
    pid = pl.program_id(0)
    dinv = dv[...]
    zL = dinv * (aL[...] + hL[...])
    zR = dinv * (aR[...] + hR[...])
    z = jnp.concatenate([zL, zR], axis=1) + b[...]
    z_ref[...] = z
    rows = pid * 256 + lax.broadcasted_iota(jnp.int32, (256, 1), 0)
    zm = jnp.where(rows < N, z, 0.0)
    s1 = jnp.sum(zm, axis=0, keepdims=True)
    s2 = jnp.sum(zm * zm, axis=0, keepdims=True)
    part = jnp.concatenate([s1, s2], axis=0)

    @pl.when(pid == 0)
    def _():
        st_ref[...] = jnp.zeros((8, D), f32)
    st_ref[0:2, :] += part


def _k2(aL, aR, hL, hR, dv, b):
    return pl.pallas_call(
        _k2_body,
        grid=(_GRID,),
        in_specs=[pl.BlockSpec((256, H), lambda i: (i, 0)),
                  pl.BlockSpec((256, H), lambda i: (i, 0)),
                  pl.BlockSpec((256, H), lambda i: (i, 0)),
                  pl.BlockSpec((256, H), lambda i: (i, 0)),
                  pl.BlockSpec((256, H), lambda i: (i, 0)),
                  pl.BlockSpec((1, D), lambda i: (0, 0))],
        out_specs=[pl.BlockSpec((256, D), lambda i: (i, 0)),
                   pl.BlockSpec((8, D), lambda i: (0, 0))],
        out_shape=[jax.ShapeDtypeStruct((NPAD, D), f32),
                   jax.ShapeDtypeStruct((8, D), f32)],
        compiler_params=pltpu.CompilerParams(
            dimension_semantics=("arbitrary",)),
    )(aL, aR, hL, hR, dv, b)


def _k1_body(z_ref, st_ref, g_ref, be_ref, w_ref, dv_ref, hl_ref, hr_ref):
    st = st_ref[...]
    mu = st[0:1, :] * (1.0 / N)
    ex2 = st[1:2, :] * (1.0 / N)
    var = ex2 - mu * mu
    scale = g_ref[...] * lax.rsqrt(var + 1e-5)
    h = jnp.maximum(scale * (z_ref[...] - mu) + be_ref[...], 0.0)
    hw = jnp.dot(h, w_ref[...], preferred_element_type=f32)
    dinv = dv_ref[...]
    hl_ref[...] = hw[:, :H] * dinv
    hr_ref[...] = hw[:, H:] * dinv


def _k1(z, st, g, be, W, dv):
    return pl.pallas_call(
        _k1_body,
        grid=(_GRID,),
        in_specs=[pl.BlockSpec((256, D), lambda i: (i, 0)),
                  pl.BlockSpec((8, D), lambda i: (0, 0)),
                  pl.BlockSpec((1, D), lambda i: (0, 0)),
                  pl.BlockSpec((1, D), lambda i: (0, 0)),
                  pl.BlockSpec((D, D), lambda i: (0, 0)),
                  pl.BlockSpec((256, H), lambda i: (i, 0))],
        out_specs=[pl.BlockSpec((256, H), lambda i: (i, 0)),
                   pl.BlockSpec((256, H), lambda i: (i, 0))],
        out_shape=[jax.ShapeDtypeStruct((NPAD, H), f32),
                   jax.ShapeDtypeStruct((NPAD, H), f32)],
    )(z, st, g, be, W, dv)


def _k1_final_body(z_ref, st_ref, g_ref, be_ref, w_ref, dv_ref,
                   ml_ref, mr_ref, ll_ref, lr_ref):
    st = st_ref[...]
    mu = st[0:1, :] * (1.0 / N)
    ex2 = st[1:2, :] * (1.0 / N)
    var = ex2 - mu * mu
    scale = g_ref[...] * lax.rsqrt(var + 1e-5)
    h = jnp.maximum(scale * (z_ref[...] - mu) + be_ref[...], 0.0)
    hw = jnp.dot(h, w_ref[...], preferred_element_type=f32)
    dinv = dv_ref[...]
    ml_ref[...] = hw[:, 0:H] * dinv
    mr_ref[...] = hw[:, H:2 * H] * dinv
    ll_ref[...] = hw[:, 2 * H:3 * H] * dinv
    lr_ref[...] = hw[:, 3 * H:] * dinv


def _k1_final(z, st, g, be, Wml, dv):
    return pl.pallas_call(
        _k1_final_body,
        grid=(_GRID,),
        in_specs=[pl.BlockSpec((256, D), lambda i: (i, 0)),
                  pl.BlockSpec((8, D), lambda i: (0, 0)),
                  pl.BlockSpec((1, D), lambda i: (0, 0)),
                  pl.BlockSpec((1, D), lambda i: (0, 0)),
                  pl.BlockSpec((D, 2 * D), lambda i: (0, 0)),
                  pl.BlockSpec((256, H), lambda i: (i, 0))],
        out_specs=[pl.BlockSpec((256, H), lambda i: (i, 0))] * 4,
        out_shape=[jax.ShapeDtypeStruct((NPAD, H), f32)] * 4,
    )(z, st, g, be, Wml, dv)


def _k2_final_body(amL, amR, alL, alR, hmL, hmR, hlL, hlR, dv, bm, bl,
                   mu_ref, ls_ref):
    dinv = dv[...]
    muL = dinv * (amL[...] + hmL[...])
    muR = dinv * (amR[...] + hmR[...])
    lsL = dinv * (alL[...] + hlL[...])
    lsR = dinv * (alR[...] + hlR[...])
    mu_ref[...] = jnp.concatenate([muL, muR], axis=1) + bm[...]
    ls_ref[...] = jnp.concatenate([lsL, lsR], axis=1) + bl[...]


def _k2_final(amL, amR, alL, alR, hmL, hmR, hlL, hlR, dv, bm, bl):
    bs = pl.BlockSpec((256, H), lambda i: (i, 0))
    bb = pl.BlockSpec((1, D), lambda i: (0, 0))
    return pl.pallas_call(
        _k2_final_body,
        grid=(_GRID,),
        in_specs=[bs] * 9 + [bb, bb],
        out_specs=[pl.BlockSpec((256, D), lambda i: (i, 0))] * 2,
        out_shape=[jax.ShapeDtypeStruct((NPAD, D), f32)] * 2,
    )(amL, amR, alL, alR, hmL, hmR, hlL, hlR, dv, bm, bl)


def kernel(x, edge_index, emb, convW, convB, bnG, bnB, Wmu, bmu, Wls, bls):
    src, dst = edge_index[0], edge_index[1]
    srcp = jnp.concatenate([src, jnp.zeros((EPAD - E,), jnp.int32)])
    dstp = jnp.concatenate([dst, jnp.full((EPAD - E,), TRASH, jnp.int32)])
    xp = jnp.concatenate([x, jnp.zeros((NPAD - N,), jnp.int32)])
    zeros1 = jnp.zeros((RPT,), f32)
    zeros_h = jnp.zeros((RPT, H), f32)

    h0, deg = _prep(xp, emb, dstp, zeros1)
    degb = jnp.broadcast_to(deg[:, None], (NPAD, H))

    hwL, hwR, dv = _k1_first(h0, convW[0], degb)
    z = None
    st = None
    for i in range(4):
        if i > 0:
            hwL, hwR = _k1(z, st, bnG[i - 1][None, :], bnB[i - 1][None, :],
                           convW[i], dv)
        aL, aR = _msg(hwL, hwR, srcp, dstp, zeros_h)
        z, st = _k2(aL, aR, hwL, hwR, dv, convB[i][None, :])

    Wml = jnp.concatenate([Wmu, Wls], axis=1)
    hmL, hmR, hlL, hlR = _k1_final(z, st, bnG[3][None, :], bnB[3][None, :],
                                   Wml, dv)
    amL, amR = _msg(hmL, hmR, srcp, dstp, zeros_h)
    alL, alR = _msg(hlL, hlR, srcp, dstp, zeros_h)
    mu_out, ls_out = _k2_final(amL, amR, alL, alR, hmL, hmR, hlL, hlR, dv,
                               bmu[None, :], bls[None, :])
    return mu_out[:N], ls_out[:N]


# speedup vs baseline: 4.0341x; 4.0341x over previous
import jax
import jax.numpy as jnp
from jax import lax
from jax.experimental import pallas as pl
from jax.experimental.pallas import tpu as pltpu, tpu_sc as plsc

N = 10000
NPAD = 10240
E = 160000
EPAD = 163840
D = 256
H = 128
TRASH = 10016
CK = 128
EPT = EPAD // 16
NCH = EPT // CK
RPT = NPAD // 16
XPT = NPAD // 32
XCK = 80

_mesh = plsc.VectorSubcoreMesh(core_axis_name="c", subcore_axis_name="s")
f32 = jnp.float32


def _prep_body(xp, emb, dstp, zeros1, h0, deg, deg_sh, rows_v, xi, di, ones_v,
               sem):
    c = lax.axis_index("c")
    s = lax.axis_index("s")
    wid = s * 2 + c
    for i in range(8):
        ones_v[pl.ds(i * 16, 16)] = jnp.ones((16,), f32)
    for j in range(XPT // XCK):
        off = wid * XPT + j * XCK
        pltpu.sync_copy(xp.at[pl.ds(off, XCK)], xi)
        pltpu.async_copy(emb.at[xi], rows_v, sem).wait()
        pltpu.sync_copy(rows_v, h0.at[pl.ds(off, XCK)])

    @pl.when(c == 0)
    def _():
        pltpu.sync_copy(zeros1.at[pl.ds(0, RPT)], deg_sh.at[pl.ds(s * RPT, RPT)])
    plsc.subcore_barrier()

    @pl.when(c == 0)
    def _():
        def body(j, carry):
            off = s * EPT + j * CK
            pltpu.sync_copy(dstp.at[pl.ds(off, CK)], di)
            pltpu.sync_copy(ones_v, deg_sh.at[di], add=True)
            return carry
        lax.fori_loop(0, NCH, body, 0)
    plsc.subcore_barrier()

    @pl.when(c == 0)
    def _():
        pltpu.sync_copy(deg_sh.at[pl.ds(s * RPT, RPT)], deg.at[pl.ds(s * RPT, RPT)])


_prep = pl.kernel(
    _prep_body,
    out_type=(jax.ShapeDtypeStruct((NPAD, D), f32),
              jax.ShapeDtypeStruct((NPAD,), f32)),
    mesh=_mesh,
    scratch_types=[pltpu.VMEM_SHARED((NPAD,), f32),
                   pltpu.VMEM((XCK, D), f32),
                   pltpu.VMEM((XCK,), jnp.int32),
                   pltpu.VMEM((CK,), jnp.int32),
                   pltpu.VMEM((CK,), f32),
                   pltpu.SemaphoreType.DMA],
)


def _msg_body(hw_flat, src2, dstp, zeros_h, acc2, acc_sh, rows_v, si, di,
              sem):
    c = lax.axis_index("c")
    s = lax.axis_index("s")
    pltpu.sync_copy(zeros_h, acc_sh.at[pl.ds(s * RPT, RPT)])
    plsc.subcore_barrier()

    def body(j, carry):
        off = s * EPT + j * CK
        pltpu.sync_copy(src2.at[pl.ds(c * EPAD + off, CK)], si)
        pltpu.sync_copy(dstp.at[pl.ds(off, CK)], di)
        pltpu.async_copy(hw_flat.at[si], rows_v, sem).wait()
        pltpu.sync_copy(rows_v, acc_sh.at[di], add=True)
        return carry
    lax.fori_loop(0, NCH, body, 0)
    plsc.subcore_barrier()
    pltpu.sync_copy(acc_sh.at[pl.ds(s * RPT, RPT)],
                    acc2.at[pl.ds(c * NPAD + s * RPT, RPT)])


_msg = pl.kernel(
    _msg_body,
    out_type=jax.ShapeDtypeStruct((2 * NPAD, H), f32),
    mesh=_mesh,
    scratch_types=[pltpu.VMEM_SHARED((NPAD, H), f32),
                   pltpu.VMEM((CK, H), f32),
                   pltpu.VMEM((CK,), jnp.int32),
                   pltpu.VMEM((CK,), jnp.int32),
                   pltpu.SemaphoreType.DMA],
)


_GRID = NPAD // 256


def _k1_first_body(h_ref, w_ref, deg_ref, hw_ref, dv_ref):
    dinv = lax.rsqrt(deg_ref[...] + 1.0)
    hw = jnp.dot(h_ref[...], w_ref[...], preferred_element_type=f32)
    dinvb = jnp.concatenate([dinv, dinv], axis=1)
    hw_ref[...] = (hw * dinvb).reshape(256, 2, H)
    dv_ref[...] = dinv


def _k1_first(h0, W0, degb):
    return pl.pallas_call(
        _k1_first_body,
        grid=(_GRID,),
        in_specs=[pl.BlockSpec((256, D), lambda i: (i, 0)),
                  pl.BlockSpec((D, D), lambda i: (0, 0)),
                  pl.BlockSpec((256, H), lambda i: (i, 0))],
        out_specs=[pl.BlockSpec((256, 2, H), lambda i: (i, 0, 0)),
                   pl.BlockSpec((256, H), lambda i: (i, 0))],
        out_shape=[jax.ShapeDtypeStruct((NPAD, 2, H), f32),
                   jax.ShapeDtypeStruct((NPAD, H), f32)],
    )(h0, W0, degb)


def _k2_body(aL, aR, h3, dv, b, z_ref, st_ref):
    pid = pl.program_id(0)
    dinv = dv[...]
    hw = h3[...]
    zL = dinv * (aL[...] + hw[:, 0, :])
    zR = dinv * (aR[...] + hw[:, 1, :])
    z = jnp.concatenate([zL, zR], axis=1) + b[...]
    z_ref[...] = z
    rows = pid * 256 + lax.broadcasted_iota(jnp.int32, (256, 1), 0)
    zm = jnp.where(rows < N, z, 0.0)
    s1 = jnp.sum(zm, axis=0, keepdims=True)
    s2 = jnp.sum(zm * zm, axis=0, keepdims=True)
    part = jnp.concatenate([jnp.broadcast_to(s1, (4, D)),
                            jnp.broadcast_to(s2, (4, D))], axis=0)

    @pl.when(pid == 0)
    def _():
        st_ref[...] = jnp.zeros((8, D), f32)
    st_ref[...] += part


def _k2(acc2, hw3, dv, b):
    return pl.pallas_call(
        _k2_body,
        grid=(_GRID,),
        in_specs=[pl.BlockSpec((256, H), lambda i: (i, 0)),
                  pl.BlockSpec((256, H), lambda i: (i + _GRID, 0)),
                  pl.BlockSpec((256, 2, H), lambda i: (i, 0, 0)),
                  pl.BlockSpec((256, H), lambda i: (i, 0)),
                  pl.BlockSpec((1, D), lambda i: (0, 0))],
        out_specs=[pl.BlockSpec((256, D), lambda i: (i, 0)),
                   pl.BlockSpec((8, D), lambda i: (0, 0))],
        out_shape=[jax.ShapeDtypeStruct((NPAD, D), f32),
                   jax.ShapeDtypeStruct((8, D), f32)],
        compiler_params=pltpu.CompilerParams(
            dimension_semantics=("arbitrary",)),
    )(acc2, acc2, hw3, dv, b)


def _bn_relu(z_ref, st_ref, g_ref, be_ref):
    st = st_ref[...]
    mu = st[0:1, :] * (1.0 / N)
    ex2 = st[4:5, :] * (1.0 / N)
    var = ex2 - mu * mu
    scale = g_ref[...] * lax.rsqrt(var + 1e-5)
    return jnp.maximum(scale * (z_ref[...] - mu) + be_ref[...], 0.0)


def _k1_body(z_ref, st_ref, g_ref, be_ref, w_ref, dv_ref, hw_ref):
    h = _bn_relu(z_ref, st_ref, g_ref, be_ref)
    hw = jnp.dot(h, w_ref[...], preferred_element_type=f32)
    dinv = dv_ref[...]
    dinvb = jnp.concatenate([dinv, dinv], axis=1)
    hw_ref[...] = (hw * dinvb).reshape(256, 2, H)


def _k1(z, st, g, be, W, dv):
    return pl.pallas_call(
        _k1_body,
        grid=(_GRID,),
        in_specs=[pl.BlockSpec((256, D), lambda i: (i, 0)),
                  pl.BlockSpec((8, D), lambda i: (0, 0)),
                  pl.BlockSpec((1, D), lambda i: (0, 0)),
                  pl.BlockSpec((1, D), lambda i: (0, 0)),
                  pl.BlockSpec((D, D), lambda i: (0, 0)),
                  pl.BlockSpec((256, H), lambda i: (i, 0))],
        out_specs=[pl.BlockSpec((256, 2, H), lambda i: (i, 0, 0))],
        out_shape=[jax.ShapeDtypeStruct((NPAD, 2, H), f32)],
    )(z, st, g, be, W, dv)[0]


def _k1_final_body(z_ref, st_ref, g_ref, be_ref, w_ref, dv_ref,
                   hm_ref, hl_ref):
    h = _bn_relu(z_ref, st_ref, g_ref, be_ref)
    hw = jnp.dot(h, w_ref[...], preferred_element_type=f32)
    dinv = dv_ref[...]
    dinvb = jnp.concatenate([dinv, dinv], axis=1)
    hm_ref[...] = (hw[:, :D] * dinvb).reshape(256, 2, H)
    hl_ref[...] = (hw[:, D:] * dinvb).reshape(256, 2, H)


def _k1_final(z, st, g, be, Wml, dv):
    return pl.pallas_call(
        _k1_final_body,
        grid=(_GRID,),
        in_specs=[pl.BlockSpec((256, D), lambda i: (i, 0)),
                  pl.BlockSpec((8, D), lambda i: (0, 0)),
                  pl.BlockSpec((1, D), lambda i: (0, 0)),
                  pl.BlockSpec((1, D), lambda i: (0, 0)),
                  pl.BlockSpec((D, 2 * D), lambda i: (0, 0)),
                  pl.BlockSpec((256, H), lambda i: (i, 0))],
        out_specs=[pl.BlockSpec((256, 2, H), lambda i: (i, 0, 0))] * 2,
        out_shape=[jax.ShapeDtypeStruct((NPAD, 2, H), f32)] * 2,
    )(z, st, g, be, Wml, dv)


def _k2_final_body(amL, amR, alL, alR, hm3, hl3, dv, bm, bl, mu_ref, ls_ref):
    dinv = dv[...]
    hm = hm3[...]
    hl = hl3[...]
    muL = dinv * (amL[...] + hm[:, 0, :])
    muR = dinv * (amR[...] + hm[:, 1, :])
    lsL = dinv * (alL[...] + hl[:, 0, :])
    lsR = dinv * (alR[...] + hl[:, 1, :])
    mu_ref[...] = jnp.concatenate([muL, muR], axis=1) + bm[...]
    ls_ref[...] = jnp.concatenate([lsL, lsR], axis=1) + bl[...]


def _k2_final(accm2, accl2, hm3, hl3, dv, bm, bl):
    bsl = pl.BlockSpec((256, H), lambda i: (i, 0))
    bsr = pl.BlockSpec((256, H), lambda i: (i + _GRID, 0))
    bs3 = pl.BlockSpec((256, 2, H), lambda i: (i, 0, 0))
    bb = pl.BlockSpec((1, D), lambda i: (0, 0))
    return pl.pallas_call(
        _k2_final_body,
        grid=(_GRID,),
        in_specs=[bsl, bsr, bsl, bsr, bs3, bs3, bsl, bb, bb],
        out_specs=[pl.BlockSpec((256, D), lambda i: (i, 0))] * 2,
        out_shape=[jax.ShapeDtypeStruct((NPAD, D), f32)] * 2,
    )(accm2, accm2, accl2, accl2, hm3, hl3, dv, bm, bl)


def kernel(x, edge_index, emb, convW, convB, bnG, bnB, Wmu, bmu, Wls, bls):
    src, dst = edge_index[0], edge_index[1]
    srcp = jnp.concatenate([src, jnp.zeros((EPAD - E,), jnp.int32)])
    dstp = jnp.concatenate([dst, jnp.full((EPAD - E,), TRASH, jnp.int32)])
    src2 = jnp.concatenate([2 * srcp, 2 * srcp + 1])
    xp = jnp.concatenate([x, jnp.zeros((NPAD - N,), jnp.int32)])
    zeros1 = jnp.zeros((RPT,), f32)
    zeros_h = jnp.zeros((RPT, H), f32)

    h0, deg = _prep(xp, emb, dstp, zeros1)
    degb = jnp.broadcast_to(deg[:, None], (NPAD, H))

    hw3, dv = _k1_first(h0, convW[0], degb)
    z = None
    st = None
    for i in range(4):
        if i > 0:
            hw3 = _k1(z, st, bnG[i - 1][None, :], bnB[i - 1][None, :],
                      convW[i], dv)
        acc2 = _msg(hw3.reshape(2 * NPAD, H), src2, dstp, zeros_h)
        z, st = _k2(acc2, hw3, dv, convB[i][None, :])

    Wml = jnp.concatenate([Wmu, Wls], axis=1)
    hm3, hl3 = _k1_final(z, st, bnG[3][None, :], bnB[3][None, :], Wml, dv)
    accm2 = _msg(hm3.reshape(2 * NPAD, H), src2, dstp, zeros_h)
    accl2 = _msg(hl3.reshape(2 * NPAD, H), src2, dstp, zeros_h)
    mu_out, ls_out = _k2_final(accm2, accl2, hm3, hl3, dv,
                               bmu[None, :], bls[None, :])
    return mu_out[:N], ls_out[:N]


# speedup vs baseline: 5.2122x; 1.2920x over previous
import jax
import jax.numpy as jnp
from jax import lax
from jax.experimental import pallas as pl
from jax.experimental.pallas import tpu as pltpu, tpu_sc as plsc

N = 10000
NPAD = 10240
E = 160000
EPAD = 163840
D = 256
H = 128
TRASH = 10016
CK = 128
EPT = EPAD // 16
NCH = EPT // CK
RPT = NPAD // 16
XPT = NPAD // 32
XCK = 80

_mesh = plsc.VectorSubcoreMesh(core_axis_name="c", subcore_axis_name="s")
f32 = jnp.float32


def _prep_body(xp, emb, dstp, zeros1, h0, deg, deg_sh, rows_v, xi, di, ones_v,
               sem):
    c = lax.axis_index("c")
    s = lax.axis_index("s")
    wid = s * 2 + c
    for i in range(8):
        ones_v[pl.ds(i * 16, 16)] = jnp.ones((16,), f32)
    for j in range(XPT // XCK):
        off = wid * XPT + j * XCK
        pltpu.sync_copy(xp.at[pl.ds(off, XCK)], xi)
        pltpu.async_copy(emb.at[xi], rows_v, sem).wait()
        pltpu.sync_copy(rows_v, h0.at[pl.ds(off, XCK)])

    pltpu.sync_copy(zeros1.at[pl.ds(0, RPT)], deg_sh.at[pl.ds(s * RPT, RPT)])
    plsc.subcore_barrier()
    ehalf = EPAD // 2

    def body(j, carry):
        off = c * ehalf + s * (ehalf // 16) + j * CK
        pltpu.sync_copy(dstp.at[pl.ds(off, CK)], di)
        pltpu.sync_copy(ones_v, deg_sh.at[di], add=True)
        return carry
    lax.fori_loop(0, ehalf // 16 // CK, body, 0)
    plsc.subcore_barrier()
    pltpu.sync_copy(deg_sh.at[pl.ds(s * RPT, RPT)],
                    deg.at[pl.ds(c * NPAD + s * RPT, RPT)])


_prep = pl.kernel(
    _prep_body,
    out_type=(jax.ShapeDtypeStruct((NPAD, D), f32),
              jax.ShapeDtypeStruct((2 * NPAD,), f32)),
    mesh=_mesh,
    scratch_types=[pltpu.VMEM_SHARED((NPAD,), f32),
                   pltpu.VMEM((XCK, D), f32),
                   pltpu.VMEM((XCK,), jnp.int32),
                   pltpu.VMEM((CK,), jnp.int32),
                   pltpu.VMEM((CK,), f32),
                   pltpu.SemaphoreType.DMA],
)


NB = 2


def _msg_body(hw_flat, src4, dst3, zeros_h, acc2, acc_sh, r0, r1,
              sidx, di0, di1, g0, g1, s0, s1, i0, i1):
    c = lax.axis_index("c")
    s = lax.axis_index("s")
    w = c * 16 + s
    pltpu.sync_copy(src4.at[pl.ds(w * NCH, NCH)], sidx)
    pltpu.sync_copy(zeros_h, acc_sh.at[pl.ds(s * RPT, RPT)])
    plsc.subcore_barrier()

    bufs = (r0, r1)
    dbufs = (di0, di1)
    gsems = (g0, g1)
    ssems = (s0, s1)
    isems = (i0, i1)

    def dload(j, b):
        pltpu.async_copy(dst3.at[pl.ds(s * NCH + j, 1)], dbufs[b],
                         isems[b])

    def gather(j, b):
        pltpu.async_copy(hw_flat.at[sidx.at[j]], bufs[b], gsems[b])

    def scat(b):
        pltpu.async_copy(bufs[b], acc_sh.at[dbufs[b].at[0]], ssems[b],
                         add=True)

    def dload_wait(b):
        pltpu.make_async_copy(dst3.at[pl.ds(0, 1)], dbufs[b],
                              isems[b]).wait()

    def gather_wait(b):
        pltpu.make_async_copy(hw_flat.at[sidx.at[0]], bufs[b],
                              gsems[b]).wait()

    def scat_wait(b):
        pltpu.make_async_copy(bufs[b], acc_sh.at[dbufs[b].at[0]],
                              ssems[b]).wait()

    for b in range(NB):
        dload(b, b)
        gather(b, b)

    def body(gi, carry):
        j0 = NB * gi
        for b in range(NB):
            gather_wait(b)
            dload_wait(b)
            scat(b)
        for b in range(NB):
            jn = j0 + b + NB
            scat_wait(b)

            @pl.when(jn < NCH)
            def _():
                dload(jn, b)
                gather(jn, b)
        return carry
    lax.fori_loop(0, NCH // NB, body, 0)
    plsc.subcore_barrier()
    pltpu.sync_copy(acc_sh.at[pl.ds(s * RPT, RPT)],
                    acc2.at[pl.ds(c * NPAD + s * RPT, RPT)])


_msg = pl.kernel(
    _msg_body,
    out_type=jax.ShapeDtypeStruct((2 * NPAD, H), f32),
    mesh=_mesh,
    scratch_types=[pltpu.VMEM_SHARED((NPAD, H), f32)]
                  + [pltpu.VMEM((128, H), f32)] * NB
                  + [pltpu.VMEM((NCH, 128), jnp.int32)]
                  + [pltpu.VMEM((1, 128), jnp.int32)] * NB
                  + [pltpu.SemaphoreType.DMA] * (3 * NB),
)


_GRID = NPAD // 256


def _k1_first_body(h_ref, w_ref, deg_ref, hw_ref, dv_ref):
    dinv = lax.rsqrt(deg_ref[...] + 1.0)
    hw = jnp.dot(h_ref[...], w_ref[...], preferred_element_type=f32)
    dinvb = jnp.concatenate([dinv, dinv], axis=1)
    hw_ref[...] = (hw * dinvb).reshape(256, 2, H)
    dv_ref[...] = dinv


def _k1_first(h0, W0, degb):
    return pl.pallas_call(
        _k1_first_body,
        grid=(_GRID,),
        in_specs=[pl.BlockSpec((256, D), lambda i: (i, 0)),
                  pl.BlockSpec((D, D), lambda i: (0, 0)),
                  pl.BlockSpec((256, H), lambda i: (i, 0))],
        out_specs=[pl.BlockSpec((256, 2, H), lambda i: (i, 0, 0)),
                   pl.BlockSpec((256, H), lambda i: (i, 0))],
        out_shape=[jax.ShapeDtypeStruct((NPAD, 2, H), f32),
                   jax.ShapeDtypeStruct((NPAD, H), f32)],
    )(h0, W0, degb)


def _k2_body(aL, aR, h3, dv, b, z_ref, st_ref):
    pid = pl.program_id(0)
    dinv = dv[...]
    hw = h3[...]
    zL = dinv * (aL[...] + hw[:, 0, :])
    zR = dinv * (aR[...] + hw[:, 1, :])
    z = jnp.concatenate([zL, zR], axis=1) + b[...]
    z_ref[...] = z
    rows = pid * 256 + lax.broadcasted_iota(jnp.int32, (256, 1), 0)
    zm = jnp.where(rows < N, z, 0.0)
    s1 = jnp.sum(zm, axis=0, keepdims=True)
    s2 = jnp.sum(zm * zm, axis=0, keepdims=True)
    part = jnp.concatenate([jnp.broadcast_to(s1, (4, D)),
                            jnp.broadcast_to(s2, (4, D))], axis=0)

    @pl.when(pid == 0)
    def _():
        st_ref[...] = jnp.zeros((8, D), f32)
    st_ref[...] += part


def _k2(acc2, hw3, dv, b):
    return pl.pallas_call(
        _k2_body,
        grid=(_GRID,),
        in_specs=[pl.BlockSpec((256, H), lambda i: (i, 0)),
                  pl.BlockSpec((256, H), lambda i: (i + _GRID, 0)),
                  pl.BlockSpec((256, 2, H), lambda i: (i, 0, 0)),
                  pl.BlockSpec((256, H), lambda i: (i, 0)),
                  pl.BlockSpec((1, D), lambda i: (0, 0))],
        out_specs=[pl.BlockSpec((256, D), lambda i: (i, 0)),
                   pl.BlockSpec((8, D), lambda i: (0, 0))],
        out_shape=[jax.ShapeDtypeStruct((NPAD, D), f32),
                   jax.ShapeDtypeStruct((8, D), f32)],
        compiler_params=pltpu.CompilerParams(
            dimension_semantics=("arbitrary",)),
    )(acc2, acc2, hw3, dv, b)


def _bn_relu(z_ref, st_ref, g_ref, be_ref):
    st = st_ref[...]
    mu = st[0:1, :] * (1.0 / N)
    ex2 = st[4:5, :] * (1.0 / N)
    var = ex2 - mu * mu
    scale = g_ref[...] * lax.rsqrt(var + 1e-5)
    return jnp.maximum(scale * (z_ref[...] - mu) + be_ref[...], 0.0)


def _k1_body(z_ref, st_ref, g_ref, be_ref, w_ref, dv_ref, hw_ref):
    h = _bn_relu(z_ref, st_ref, g_ref, be_ref)
    hw = jnp.dot(h, w_ref[...], preferred_element_type=f32)
    dinv = dv_ref[...]
    dinvb = jnp.concatenate([dinv, dinv], axis=1)
    hw_ref[...] = (hw * dinvb).reshape(256, 2, H)


def _k1(z, st, g, be, W, dv):
    return pl.pallas_call(
        _k1_body,
        grid=(_GRID,),
        in_specs=[pl.BlockSpec((256, D), lambda i: (i, 0)),
                  pl.BlockSpec((8, D), lambda i: (0, 0)),
                  pl.BlockSpec((1, D), lambda i: (0, 0)),
                  pl.BlockSpec((1, D), lambda i: (0, 0)),
                  pl.BlockSpec((D, D), lambda i: (0, 0)),
                  pl.BlockSpec((256, H), lambda i: (i, 0))],
        out_specs=[pl.BlockSpec((256, 2, H), lambda i: (i, 0, 0))],
        out_shape=[jax.ShapeDtypeStruct((NPAD, 2, H), f32)],
    )(z, st, g, be, W, dv)[0]


def _k1_final_body(z_ref, st_ref, g_ref, be_ref, w_ref, dv_ref,
                   hm_ref, hl_ref):
    h = _bn_relu(z_ref, st_ref, g_ref, be_ref)
    hw = jnp.dot(h, w_ref[...], preferred_element_type=f32)
    dinv = dv_ref[...]
    dinvb = jnp.concatenate([dinv, dinv], axis=1)
    hm_ref[...] = (hw[:, :D] * dinvb).reshape(256, 2, H)
    hl_ref[...] = (hw[:, D:] * dinvb).reshape(256, 2, H)


def _k1_final(z, st, g, be, Wml, dv):
    return pl.pallas_call(
        _k1_final_body,
        grid=(_GRID,),
        in_specs=[pl.BlockSpec((256, D), lambda i: (i, 0)),
                  pl.BlockSpec((8, D), lambda i: (0, 0)),
                  pl.BlockSpec((1, D), lambda i: (0, 0)),
                  pl.BlockSpec((1, D), lambda i: (0, 0)),
                  pl.BlockSpec((D, 2 * D), lambda i: (0, 0)),
                  pl.BlockSpec((256, H), lambda i: (i, 0))],
        out_specs=[pl.BlockSpec((256, 2, H), lambda i: (i, 0, 0))] * 2,
        out_shape=[jax.ShapeDtypeStruct((NPAD, 2, H), f32)] * 2,
    )(z, st, g, be, Wml, dv)


def _k2_final_body(amL, amR, alL, alR, hm3, hl3, dv, bm, bl, mu_ref, ls_ref):
    dinv = dv[...]
    hm = hm3[...]
    hl = hl3[...]
    muL = dinv * (amL[...] + hm[:, 0, :])
    muR = dinv * (amR[...] + hm[:, 1, :])
    lsL = dinv * (alL[...] + hl[:, 0, :])
    lsR = dinv * (alR[...] + hl[:, 1, :])
    mu_ref[...] = jnp.concatenate([muL, muR], axis=1) + bm[...]
    ls_ref[...] = jnp.concatenate([lsL, lsR], axis=1) + bl[...]


def _k2_final(accm2, accl2, hm3, hl3, dv, bm, bl):
    bsl = pl.BlockSpec((256, H), lambda i: (i, 0))
    bsr = pl.BlockSpec((256, H), lambda i: (i + _GRID, 0))
    bs3 = pl.BlockSpec((256, 2, H), lambda i: (i, 0, 0))
    bb = pl.BlockSpec((1, D), lambda i: (0, 0))
    return pl.pallas_call(
        _k2_final_body,
        grid=(_GRID,),
        in_specs=[bsl, bsr, bsl, bsr, bs3, bs3, bsl, bb, bb],
        out_specs=[pl.BlockSpec((256, D), lambda i: (i, 0))] * 2,
        out_shape=[jax.ShapeDtypeStruct((NPAD, D), f32)] * 2,
    )(accm2, accm2, accl2, accl2, hm3, hl3, dv, bm, bl)


def kernel(x, edge_index, emb, convW, convB, bnG, bnB, Wmu, bmu, Wls, bls):
    src, dst = edge_index[0], edge_index[1]
    srcp = jnp.concatenate([src, jnp.zeros((EPAD - E,), jnp.int32)])
    dstp = jnp.concatenate([dst, jnp.full((EPAD - E,), TRASH, jnp.int32)])
    src4 = jnp.concatenate([2 * srcp, 2 * srcp + 1]).reshape(32 * (EPT // 128), 128)
    xp = jnp.concatenate([x, jnp.zeros((NPAD - N,), jnp.int32)])
    zeros1 = jnp.zeros((RPT,), f32)
    zeros_h = jnp.zeros((RPT, H), f32)

    dst3 = dstp.reshape(16 * (EPT // 128), 128)
    h0, deg2 = _prep(xp, emb, dstp, zeros1)
    degb = jnp.broadcast_to((deg2[:NPAD] + deg2[NPAD:])[:, None], (NPAD, H))

    hw3, dv = _k1_first(h0, convW[0], degb)
    z = None
    st = None
    for i in range(4):
        if i > 0:
            hw3 = _k1(z, st, bnG[i - 1][None, :], bnB[i - 1][None, :],
                      convW[i], dv)
        acc2 = _msg(hw3.reshape(2 * NPAD, H), src4, dst3, zeros_h)
        z, st = _k2(acc2, hw3, dv, convB[i][None, :])

    Wml = jnp.concatenate([Wmu, Wls], axis=1)
    hm3, hl3 = _k1_final(z, st, bnG[3][None, :], bnB[3][None, :], Wml, dv)
    accm2 = _msg(hm3.reshape(2 * NPAD, H), src4, dst3, zeros_h)
    accl2 = _msg(hl3.reshape(2 * NPAD, H), src4, dst3, zeros_h)
    mu_out, ls_out = _k2_final(accm2, accl2, hm3, hl3, dv,
                               bmu[None, :], bls[None, :])
    return mu_out[:N], ls_out[:N]
